# Initial kernel scaffold; baseline (speedup 1.0000x reference)
import functools

import jax
import jax.numpy as jnp
import numpy as np
from jax import lax
from jax.experimental import pallas as pl
from jax.experimental.pallas import tpu as pltpu

G = 2000
NPG = 50
N = G * NPG
K1 = 5
EPPI = 64000
NTRAIN = 10000
BN_SCALE = 1.0 / np.sqrt(1.0 + 1e-5)

GB = 100
RB = 400


def _bmm(a, b, adim, bdim):
    return lax.dot_general(
        a, b, dimension_numbers=(((adim,), (bdim,)), ((0,), (0,))),
        preferred_element_type=jnp.float32)


def _graph_pipeline_body(at_ref, x_ref,
                         w1, b1, s1_, t1, w2, b2, s2_, t2, w3, b3, s3_, t3,
                         wr, br, wrt,
                         w4, b4, s4_, t4, w5, b5,
                         wr2, br2, wrt2,
                         out_ref):
    at = at_ref[...].astype(jnp.float32)
    deg = jnp.sum(at, axis=2) + 1.0
    dinv = lax.rsqrt(deg)
    x = x_ref[...]

    def conv(x, adj, dv, W, b):
        gb, n, fi = x.shape
        z = jnp.dot(x.reshape(gb * n, fi), W[...],
                    preferred_element_type=jnp.float32).reshape(gb, n, -1)
        zh = dv[..., None] * z
        mixed = _bmm(adj, zh, 2, 1)
        return dv[..., None] * (mixed + zh) + b[...][0][None, None, :]

    def bn_tanh(x, s, t):
        return jnp.tanh(x * s[...][0][None, None, :] + t[...][0][None, None, :])

    x = bn_tanh(conv(x, at, dinv, w1, b1), s1_, t1)
    x = bn_tanh(conv(x, at, dinv, w2, b2), s2_, t2)
    x = bn_tanh(conv(x, at, dinv, w3, b3), s3_, t3)

    v = jnp.sum(x * wr[...][0][None, None, :], axis=2)
    agg = jnp.sum(at * v[:, None, :], axis=2)
    vroot = jnp.sum(x * wrt[...][0][None, None, :], axis=2)
    score = jnp.tanh(agg + vroot + br[...][0, 0])

    iota = lax.broadcasted_iota(jnp.int32, score.shape, 1)
    scur = score
    ohs = []
    for _ in range(K1):
        m = jnp.max(scur, axis=1)
        idx = jnp.min(jnp.where(scur == m[:, None], iota, NPG), axis=1)
        oh = (iota == idx[:, None]).astype(jnp.float32)
        ohs.append(oh[:, None, :])
        scur = jnp.where(oh > 0.5, -1e9, scur)
    P5 = jnp.concatenate(ohs, axis=1)
    ssel = jnp.sum(P5 * score[:, None, :], axis=2)

    x1 = _bmm(P5, x, 2, 1) * ssel[..., None]
    T1 = _bmm(at, P5, 2, 2)
    a2t = _bmm(P5, T1, 2, 1)
    deg2 = jnp.sum(a2t, axis=2) + 1.0
    dinv2 = lax.rsqrt(deg2)

    x1 = bn_tanh(conv(x1, a2t, dinv2, w4, b4), s4_, t4)
    x1 = conv(x1, a2t, dinv2, w5, b5)

    v2 = jnp.sum(x1 * wr2[...][0][None, None, :], axis=2)
    agg2 = jnp.sum(a2t * v2[:, None, :], axis=2)
    vroot2 = jnp.sum(x1 * wrt2[...][0][None, None, :], axis=2)
    score2 = jnp.tanh(agg2 + vroot2 + br2[...][0, 0])

    iota5 = lax.broadcasted_iota(jnp.int32, score2.shape, 1)
    m2 = jnp.max(score2, axis=1)
    i1 = jnp.min(jnp.where(score2 == m2[:, None], iota5, K1), axis=1)
    oh1 = (iota5 == i1[:, None]).astype(jnp.float32)
    out_ref[...] = jnp.sum(x1 * (oh1 * score2)[..., None], axis=1)


def _graph_pipeline(at, x, p):
    def row(a):
        return jnp.asarray(a, jnp.float32).reshape(1, -1)

    w1, cb1 = p['conv1']
    w2, cb2 = p['conv2']
    w3, cb3 = p['conv3']
    w4, cb4 = p['conv4']
    w5, cb5 = p['conv5']
    bns = []
    for nm in ['bn1', 'bn2', 'bn3', 'bn4']:
        g, b = p[nm]
        bns += [row(g * BN_SCALE), row(b)]
    sp, sp2 = p['sag1'], p['sag2']
    ins = [
        at, x,
        w1, row(cb1), bns[0], bns[1],
        w2, row(cb2), bns[2], bns[3],
        w3, row(cb3), bns[4], bns[5],
        row(sp['Wr'][:, 0]), row(sp['br']), row(sp['Wroot'][:, 0]),
        w4, row(cb4), bns[6], bns[7],
        w5, row(cb5),
        row(sp2['Wr'][:, 0]), row(sp2['br']), row(sp2['Wroot'][:, 0]),
    ]
    full = lambda a: pl.BlockSpec(a.shape, lambda i: (0,) * a.ndim)
    in_specs = [
        pl.BlockSpec((GB, NPG, NPG), lambda i: (i, 0, 0)),
        pl.BlockSpec((GB, NPG, 7), lambda i: (i, 0, 0)),
    ] + [full(a) for a in ins[2:]]
    return pl.pallas_call(
        _graph_pipeline_body,
        grid=(G // GB,),
        in_specs=in_specs,
        out_specs=pl.BlockSpec((GB, 64), lambda i: (i, 0)),
        out_shape=jax.ShapeDtypeStruct((G, 64), jnp.float32),
    )(*ins)


def _gin_a_body(b_ref, embs_ref, eps, w1, b1, w2, b2, w3, b3, bs, bb, out_ref):
    bmat = b_ref[...].astype(jnp.float32)
    embs = embs_ref[...]
    i = pl.program_id(0)
    own = lax.dynamic_slice_in_dim(embs, i * RB, RB)
    h = (1.0 + eps[...][0, 0]) * own + jnp.dot(
        bmat, embs, preferred_element_type=jnp.float32)
    mm = lambda a, w: jnp.dot(a, w[...], preferred_element_type=jnp.float32)
    h = jax.nn.relu(mm(h, w1) + b1[...][0][None, :])
    h = jax.nn.relu(mm(h, w2) + b2[...][0][None, :])
    h = jax.nn.relu(mm(h, w3) + b3[...][0][None, :])
    out_ref[...] = h * bs[...][0][None, :] + bb[...][0][None, :]


def _gin_b_body(b_ref, h_ref, eps, w, b, bs, bb, wl1, bl1, wl2, bl2, out_ref):
    bmat = b_ref[...].astype(jnp.float32)
    h3 = h_ref[...]
    i = pl.program_id(0)
    own = lax.dynamic_slice_in_dim(h3, i * RB, RB)
    h = (1.0 + eps[...][0, 0]) * own + jnp.dot(
        bmat, h3, preferred_element_type=jnp.float32)
    mm = lambda a, w_: jnp.dot(a, w_[...], preferred_element_type=jnp.float32)
    h = jax.nn.relu(mm(h, w) + b[...][0][None, :])
    h = h * bs[...][0][None, :] + bb[...][0][None, :]
    h = jax.nn.relu(mm(h, wl1) + bl1[...][0][None, :])
    out_ref[...] = mm(h, wl2) + bl2[...][0][None, :]


def _gin(bmat, embs, p):
    row = lambda a: jnp.asarray(a, jnp.float32).reshape(1, -1)
    gp, gp2 = p['gin1'], p['gin2']
    wl1, bl1 = p['lin1']
    wl2, bl2 = p['lin2']
    full = lambda a: pl.BlockSpec(a.shape, lambda i: (0,) * a.ndim)

    ins_a = [bmat, embs, row(gp['eps']), gp['W1'], row(gp['b1']),
             gp['W2'], row(gp['b2']), gp['W3'], row(gp['b3']),
             row(gp['bng'] * BN_SCALE), row(gp['bnb'])]
    h3 = pl.pallas_call(
        _gin_a_body,
        grid=(G // RB,),
        in_specs=[pl.BlockSpec((RB, G), lambda i: (i, 0))] + [full(a) for a in ins_a[1:]],
        out_specs=pl.BlockSpec((RB, 512), lambda i: (i, 0)),
        out_shape=jax.ShapeDtypeStruct((G, 512), jnp.float32),
    )(*ins_a)

    ins_b = [bmat, h3, row(gp2['eps']), gp2['W'], row(gp2['b']),
             row(gp2['bng'] * BN_SCALE), row(gp2['bnb']),
             wl1, row(bl1), wl2, row(bl2)]
    return pl.pallas_call(
        _gin_b_body,
        grid=(G // RB,),
        in_specs=[pl.BlockSpec((RB, G), lambda i: (i, 0))] + [full(a) for a in ins_b[1:]],
        out_specs=pl.BlockSpec((RB, 512), lambda i: (i, 0)),
        out_shape=jax.ShapeDtypeStruct((G, 512), jnp.float32),
    )(*ins_b)


def _fc2_body(hm_ref, w_ref, b_ref, out_ref):
    out_ref[...] = (jnp.dot(hm_ref[...], w_ref[...],
                            preferred_element_type=jnp.float32)
                    + b_ref[...][0][None, :])


def _fc2(hm, w, b):
    nb = hm.shape[0] // 1000
    return pl.pallas_call(
        _fc2_body,
        grid=(nb,),
        in_specs=[pl.BlockSpec((1000, 512), lambda i: (i, 0)),
                  pl.BlockSpec(w.shape, lambda i: (0, 0)),
                  pl.BlockSpec((1, 7), lambda i: (0, 0))],
        out_specs=pl.BlockSpec((1000, 7), lambda i: (i, 0)),
        out_shape=jax.ShapeDtypeStruct((hm.shape[0], 7), jnp.float32),
    )(hm, w, b.reshape(1, -1))


def _build_adj_placeholder(p_edge_all, edge_index):
    src, dst = p_edge_all[0], p_edge_all[1]
    flat = dst * NPG + src % NPG
    at = jnp.zeros((N * NPG,), jnp.float32).at[flat].add(1.0)
    es, ed = edge_index[0], edge_index[1]
    b = jnp.zeros((G * G,), jnp.float32).at[ed * G + es].add(1.0)
    return (at.reshape(G, NPG, NPG).astype(jnp.bfloat16),
            b.reshape(G, G).astype(jnp.bfloat16))


def _pair_gather_placeholder(h, edge_index, train_edge_id):
    node_id = edge_index[:, train_edge_id]
    return h[node_id[0]] * h[node_id[1]]


def kernel(batch, p_x_all, p_edge_all, edge_index, train_edge_id, params):
    at, bmat = _build_adj_placeholder(p_edge_all, edge_index)
    embs = _graph_pipeline(at, p_x_all.reshape(G, NPG, 7), params)
    h = _gin(bmat, embs, params)
    hm = _pair_gather_placeholder(h, edge_index, train_edge_id)
    w, b = params['fc2']
    return _fc2(hm, w, b)


# speedup vs baseline: 23.0868x; 23.0868x over previous
import functools

import jax
import jax.numpy as jnp
import numpy as np
from jax import lax
from jax.experimental import pallas as pl
from jax.experimental.pallas import tpu as pltpu
from jax.experimental.pallas import tpu_sc as plsc

G = 2000
NPG = 50
N = G * NPG
K1 = 5
EPPI = 64000
NTRAIN = 10000
BN_SCALE = 1.0 / np.sqrt(1.0 + 1e-5)

GB = 40
RB = 400

HI = lax.Precision.HIGHEST


def _bmm(a, b, adim, bdim, prec=None):
    return lax.dot_general(
        a, b, dimension_numbers=(((adim,), (bdim,)), ((0,), (0,))),
        preferred_element_type=jnp.float32, precision=prec)


def _graph_pipeline_body(at_ref, x_ref,
                         w1, b1, s1_, t1, w2, b2, s2_, t2, w3, b3, s3_, t3,
                         wr, br, wrt,
                         w4, b4, s4_, t4, w5, b5,
                         wr2, br2, wrt2,
                         out_ref):
    at = at_ref[...].astype(jnp.float32)
    deg = jnp.sum(at, axis=2) + 1.0
    dinv = lax.rsqrt(deg)
    x = x_ref[...]

    def conv(x, adj, dv, W, b):
        gb, n, fi = x.shape
        z = jnp.dot(x.reshape(gb * n, fi), W[...],
                    preferred_element_type=jnp.float32).reshape(gb, n, -1)
        zh = dv[..., None] * z
        mixed = _bmm(adj, zh, 2, 1, HI)
        return dv[..., None] * (mixed + zh) + b[...][0][None, None, :]

    def bn_tanh(x, s, t):
        return jnp.tanh(x * s[...][0][None, None, :] + t[...][0][None, None, :])

    x = bn_tanh(conv(x, at, dinv, w1, b1), s1_, t1)
    x = bn_tanh(conv(x, at, dinv, w2, b2), s2_, t2)
    x = bn_tanh(conv(x, at, dinv, w3, b3), s3_, t3)

    xf = x.reshape(GB * NPG, 32)
    v = jnp.dot(xf, wr[...][0][:, None],
                preferred_element_type=jnp.float32).reshape(GB, NPG)
    agg = jnp.sum(at * v[:, None, :], axis=2)
    vroot = jnp.dot(xf, wrt[...][0][:, None],
                    preferred_element_type=jnp.float32).reshape(GB, NPG)
    score = jnp.tanh(agg + vroot + br[...][0, 0])

    iota = lax.broadcasted_iota(jnp.int32, score.shape, 1)
    scur = score
    ohs = []
    for _ in range(K1):
        m = jnp.max(scur, axis=1)
        idx = jnp.min(jnp.where(scur == m[:, None], iota, NPG), axis=1)
        oh = (iota == idx[:, None]).astype(jnp.float32)
        ohs.append(oh[:, None, :])
        scur = jnp.where(oh > 0.5, -1e9, scur)
    P5 = jnp.concatenate(ohs, axis=1)
    ssel = jnp.sum(P5 * score[:, None, :], axis=2)

    x1 = _bmm(P5, x, 2, 1, HI) * ssel[..., None]
    T1 = _bmm(at, P5, 2, 2)
    a2t = _bmm(P5, T1, 2, 1)
    deg2 = jnp.sum(a2t, axis=2) + 1.0
    dinv2 = lax.rsqrt(deg2)

    x1 = bn_tanh(conv(x1, a2t, dinv2, w4, b4), s4_, t4)
    x1 = conv(x1, a2t, dinv2, w5, b5)

    x1f = x1.reshape(GB * K1, 64)
    v2 = jnp.dot(x1f, wr2[...][0][:, None],
                 preferred_element_type=jnp.float32).reshape(GB, K1)
    agg2 = jnp.sum(a2t * v2[:, None, :], axis=2)
    vroot2 = jnp.dot(x1f, wrt2[...][0][:, None],
                     preferred_element_type=jnp.float32).reshape(GB, K1)
    score2 = jnp.tanh(agg2 + vroot2 + br2[...][0, 0])

    iota5 = lax.broadcasted_iota(jnp.int32, score2.shape, 1)
    m2 = jnp.max(score2, axis=1)
    i1 = jnp.min(jnp.where(score2 == m2[:, None], iota5, K1), axis=1)
    oh1 = (iota5 == i1[:, None]).astype(jnp.float32)
    out_ref[...] = jnp.sum(x1 * (oh1 * score2)[..., None], axis=1)


def _graph_pipeline(at, x, p):
    def row(a):
        return jnp.asarray(a, jnp.float32).reshape(1, -1)

    w1, cb1 = p['conv1']
    w2, cb2 = p['conv2']
    w3, cb3 = p['conv3']
    w4, cb4 = p['conv4']
    w5, cb5 = p['conv5']
    bns = []
    for nm in ['bn1', 'bn2', 'bn3', 'bn4']:
        g, b = p[nm]
        bns += [row(g * BN_SCALE), row(b)]
    sp, sp2 = p['sag1'], p['sag2']
    ins = [
        at, x,
        w1, row(cb1), bns[0], bns[1],
        w2, row(cb2), bns[2], bns[3],
        w3, row(cb3), bns[4], bns[5],
        row(sp['Wr'][:, 0]), row(sp['br']), row(sp['Wroot'][:, 0]),
        w4, row(cb4), bns[6], bns[7],
        w5, row(cb5),
        row(sp2['Wr'][:, 0]), row(sp2['br']), row(sp2['Wroot'][:, 0]),
    ]
    full = lambda a: pl.BlockSpec(a.shape, lambda i: (0,) * a.ndim)
    in_specs = [
        pl.BlockSpec((GB, NPG, NPG), lambda i: (i, 0, 0)),
        pl.BlockSpec((GB, NPG, 7), lambda i: (i, 0, 0)),
    ] + [full(a) for a in ins[2:]]
    return pl.pallas_call(
        _graph_pipeline_body,
        grid=(G // GB,),
        in_specs=in_specs,
        out_specs=pl.BlockSpec((GB, 64), lambda i: (i, 0)),
        out_shape=jax.ShapeDtypeStruct((G, 64), jnp.float32),
    )(*ins)


def _gin_a_body(b_ref, embs_ref, own_ref, eps, w1, b1, w2, b2, w3, b3, bs, bb,
                out_ref):
    bmat = b_ref[...].astype(jnp.float32)
    embs = embs_ref[...]
    own = own_ref[...]
    h = (1.0 + eps[...][0, 0]) * own + jnp.dot(
        bmat, embs, preferred_element_type=jnp.float32, precision=HI)
    mm = lambda a, w: jnp.dot(a, w[...], preferred_element_type=jnp.float32)
    h = jax.nn.relu(mm(h, w1) + b1[...][0][None, :])
    h = jax.nn.relu(mm(h, w2) + b2[...][0][None, :])
    h = jax.nn.relu(mm(h, w3) + b3[...][0][None, :])
    out_ref[...] = h * bs[...][0][None, :] + bb[...][0][None, :]


def _gin_b_body(b_ref, h_ref, own_ref, eps, w, b, bs, bb, wl1, bl1, wl2, bl2,
                out_ref):
    bmat = b_ref[...].astype(jnp.float32)
    h3 = h_ref[...]
    own = own_ref[...]
    h = (1.0 + eps[...][0, 0]) * own + jnp.dot(
        bmat, h3, preferred_element_type=jnp.float32, precision=HI)
    mm = lambda a, w_: jnp.dot(a, w_[...], preferred_element_type=jnp.float32)
    h = jax.nn.relu(mm(h, w) + b[...][0][None, :])
    h = h * bs[...][0][None, :] + bb[...][0][None, :]
    h = jax.nn.relu(mm(h, wl1) + bl1[...][0][None, :])
    out_ref[...] = mm(h, wl2) + bl2[...][0][None, :]


def _gin(bmat, embs, p):
    row = lambda a: jnp.asarray(a, jnp.float32).reshape(1, -1)
    gp, gp2 = p['gin1'], p['gin2']
    wl1, bl1 = p['lin1']
    wl2, bl2 = p['lin2']
    full = lambda a: pl.BlockSpec(a.shape, lambda i: (0,) * a.ndim)

    ins_a = [bmat, embs, embs, row(gp['eps']), gp['W1'], row(gp['b1']),
             gp['W2'], row(gp['b2']), gp['W3'], row(gp['b3']),
             row(gp['bng'] * BN_SCALE), row(gp['bnb'])]
    h3 = pl.pallas_call(
        _gin_a_body,
        grid=(G // RB,),
        in_specs=[pl.BlockSpec((RB, G), lambda i: (i, 0)),
                  full(embs),
                  pl.BlockSpec((RB, 64), lambda i: (i, 0))] + [full(a) for a in ins_a[3:]],
        out_specs=pl.BlockSpec((RB, 512), lambda i: (i, 0)),
        out_shape=jax.ShapeDtypeStruct((G, 512), jnp.float32),
    )(*ins_a)

    ins_b = [bmat, h3, h3, row(gp2['eps']), gp2['W'], row(gp2['b']),
             row(gp2['bng'] * BN_SCALE), row(gp2['bnb']),
             wl1, row(bl1), wl2, row(bl2)]
    return pl.pallas_call(
        _gin_b_body,
        grid=(G // RB,),
        in_specs=[pl.BlockSpec((RB, G), lambda i: (i, 0)),
                  full(h3),
                  pl.BlockSpec((RB, 512), lambda i: (i, 0))] + [full(a) for a in ins_b[3:]],
        out_specs=pl.BlockSpec((RB, 512), lambda i: (i, 0)),
        out_shape=jax.ShapeDtypeStruct((G, 512), jnp.float32),
    )(*ins_b)


def _fc2_body(x1_ref, x2_ref, w_ref, b_ref, out_ref):
    out_ref[...] = (jnp.dot(x1_ref[...] * x2_ref[...], w_ref[...],
                            preferred_element_type=jnp.float32)
                    + b_ref[...][0][None, :])


def _fc2(x1, x2, w, b):
    nb = x1.shape[0] // 1024
    return pl.pallas_call(
        _fc2_body,
        grid=(nb,),
        in_specs=[pl.BlockSpec((1024, 512), lambda i: (i, 0)),
                  pl.BlockSpec((1024, 512), lambda i: (i, 0)),
                  pl.BlockSpec(w.shape, lambda i: (0, 0)),
                  pl.BlockSpec((1, 7), lambda i: (0, 0))],
        out_specs=pl.BlockSpec((1024, 7), lambda i: (i, 0)),
        out_shape=jax.ShapeDtypeStruct((x1.shape[0], 7), jnp.float32),
    )(x1, x2, w, b.reshape(1, -1))


EPAD = 819200
PPAD = 65536
AHALF = N * NPG // 2
BHALF = G * G // 2
SH = 2506752
ZCH = 156672
CH = 2048


def _adj_sc_body(src_hbm, dst_hbm, es_hbm, ed_hbm, zeros_hbm, ones_hbm,
                 at_out, b_out,
                 e0_buf, e1_buf, idx_buf, ones_buf, shared, sem):
    c = lax.axis_index("c")
    s = lax.axis_index("s")
    cb = c * AHALF
    pb = c * BHALF

    pltpu.sync_copy(ones_hbm, ones_buf)

    def zero_shared():
        soff = pl.multiple_of(s * ZCH, 256)
        pltpu.sync_copy(zeros_hbm, shared.at[pl.ds(soff, ZCH)])

    def scatter_edges(src_ref, dst_ref, nchunks, per_sub, base, half, scale):
        basev = jnp.full((16,), base, jnp.int32)

        def chunk(ci, _):
            e0 = pl.multiple_of(s * per_sub + ci * CH, 256)
            pltpu.sync_copy(src_ref.at[pl.ds(e0, CH)], e0_buf)
            pltpu.sync_copy(dst_ref.at[pl.ds(e0, CH)], e1_buf)

            for j in range(16):
                for v in range(8):
                    off = j * 128 + v * 16
                    sv = e0_buf[pl.ds(off, 16)]
                    dv = e1_buf[pl.ds(off, 16)]
                    if scale == NPG:
                        col = sv - (sv // NPG) * NPG
                    else:
                        col = sv
                    rel = dv * scale + col - basev
                    ok = (rel >= 0) & (rel < half)
                    pos = lax.iota(jnp.int32, 16) + off
                    dummy = half + (pos & 4095)
                    idx_buf[j, pl.ds(v * 16, 16)] = jnp.where(ok, rel, dummy)

            for j in range(16):
                pltpu.sync_copy(ones_buf, shared.at[idx_buf.at[j]], add=True)
            return 0

        lax.fori_loop(0, nchunks, chunk, 0)

    def copy_out(out_ref):
        soff = pl.multiple_of(s * ZCH, 256)
        ooff = pl.multiple_of(c * SH + s * ZCH, 256)
        pltpu.sync_copy(shared.at[pl.ds(soff, ZCH)],
                        out_ref.at[pl.ds(ooff, ZCH)])

    zero_shared()
    plsc.subcore_barrier()
    scatter_edges(src_hbm, dst_hbm, EPAD // (16 * CH), EPAD // 16,
                  cb, AHALF, NPG)
    plsc.subcore_barrier()
    copy_out(at_out)
    plsc.subcore_barrier()
    zero_shared()
    plsc.subcore_barrier()
    scatter_edges(es_hbm, ed_hbm, PPAD // (16 * CH), PPAD // 16,
                  pb, BHALF, G)
    plsc.subcore_barrier()
    copy_out(b_out)


def _build_adj_sc(p_edge_all, edge_index):
    pad_e = jnp.full((EPAD - p_edge_all.shape[1],), N, jnp.int32)
    src = jnp.concatenate([p_edge_all[0].astype(jnp.int32), pad_e])
    dst = jnp.concatenate([p_edge_all[1].astype(jnp.int32), pad_e])
    pad_p = jnp.full((PPAD - edge_index.shape[1],), G, jnp.int32)
    es = jnp.concatenate([edge_index[0].astype(jnp.int32), pad_p])
    ed = jnp.concatenate([edge_index[1].astype(jnp.int32), pad_p])
    zeros = jnp.zeros((ZCH,), jnp.bfloat16)
    ones = jnp.ones((128,), jnp.bfloat16)

    mesh = plsc.VectorSubcoreMesh(core_axis_name="c", subcore_axis_name="s")
    at_raw, b_raw = pl.kernel(
        _adj_sc_body,
        mesh=mesh,
        out_type=[jax.ShapeDtypeStruct((2 * SH,), jnp.bfloat16),
                  jax.ShapeDtypeStruct((2 * SH,), jnp.bfloat16)],
        scratch_types=[pltpu.VMEM((CH,), jnp.int32),
                       pltpu.VMEM((CH,), jnp.int32),
                       pltpu.VMEM((16, 128), jnp.int32),
                       pltpu.VMEM((128,), jnp.bfloat16),
                       pltpu.VMEM_SHARED((SH,), jnp.bfloat16),
                       pltpu.SemaphoreType.DMA],
    )(src, dst, es, ed, zeros, ones)
    at_flat = jnp.concatenate([at_raw[:AHALF], at_raw[SH:SH + AHALF]])
    b_flat = jnp.concatenate([b_raw[:BHALF], b_raw[SH:SH + BHALF]])
    return at_flat.reshape(G, NPG, NPG), b_flat.reshape(G, G)


TPAD = 10240


def _pair_sc_body(es_hbm, ed_hbm, tid_hbm, h_hbm, x1_out, x2_out,
                  tid_buf, i0_buf, i1_buf, x1_buf, x2_buf, sem):
    c = lax.axis_index("c")
    s = lax.axis_index("s")
    wid = c * 16 + s

    def do_chunk(cbase):
        pltpu.sync_copy(tid_hbm.at[pl.ds(cbase, 256)], tid_buf)
        for k in range(4):
            idx = tid_buf.at[pl.ds(k * 64, 64)]
            pltpu.async_copy(es_hbm.at[idx], i0_buf, sem).wait()
            pltpu.async_copy(ed_hbm.at[idx], i1_buf, sem).wait()
            pltpu.async_copy(h_hbm.at[i0_buf], x1_buf, sem).wait()
            pltpu.async_copy(h_hbm.at[i1_buf], x2_buf, sem).wait()
            rb = pl.multiple_of(cbase + k * 64, 64)
            pltpu.sync_copy(x1_buf, x1_out.at[pl.ds(rb, 64), :])
            pltpu.sync_copy(x2_buf, x2_out.at[pl.ds(rb, 64), :])

    do_chunk(pl.multiple_of(wid * 256, 256))

    @pl.when(wid < (TPAD // 256) - 32)
    def _():
        do_chunk(pl.multiple_of((wid + 32) * 256, 256))


def _pair_gather_sc(h, edge_index, train_edge_id):
    tid = jnp.concatenate([
        train_edge_id.astype(jnp.int32),
        jnp.zeros((TPAD - NTRAIN,), jnp.int32)])
    es = edge_index[0].astype(jnp.int32)
    ed = edge_index[1].astype(jnp.int32)
    mesh = plsc.VectorSubcoreMesh(core_axis_name="c", subcore_axis_name="s")
    x1, x2 = pl.kernel(
        _pair_sc_body,
        mesh=mesh,
        out_type=[jax.ShapeDtypeStruct((TPAD, 512), jnp.float32),
                  jax.ShapeDtypeStruct((TPAD, 512), jnp.float32)],
        scratch_types=[pltpu.VMEM((256,), jnp.int32),
                       pltpu.VMEM((64,), jnp.int32),
                       pltpu.VMEM((64,), jnp.int32),
                       pltpu.VMEM((64, 512), jnp.float32),
                       pltpu.VMEM((64, 512), jnp.float32),
                       pltpu.SemaphoreType.DMA],
    )(es, ed, tid, h)
    return x1, x2


def _build_adj_placeholder(p_edge_all, edge_index):
    src, dst = p_edge_all[0], p_edge_all[1]
    flat = dst * NPG + src % NPG
    at = jnp.zeros((N * NPG,), jnp.float32).at[flat].add(1.0)
    es, ed = edge_index[0], edge_index[1]
    b = jnp.zeros((G * G,), jnp.float32).at[ed * G + es].add(1.0)
    return (at.reshape(G, NPG, NPG).astype(jnp.bfloat16),
            b.reshape(G, G).astype(jnp.bfloat16))


def kernel(batch, p_x_all, p_edge_all, edge_index, train_edge_id, params):
    at, bmat = _build_adj_placeholder(p_edge_all, edge_index)
    embs = _graph_pipeline(at, p_x_all.reshape(G, NPG, 7), params)
    h = _gin(bmat, embs, params)
    x1, x2 = _pair_gather_sc(h, edge_index, train_edge_id)
    w, b = params['fc2']
    return _fc2(x1, x2, w, b)[:NTRAIN]
